# Initial kernel scaffold; baseline (speedup 1.0000x reference)
import functools

import jax
import jax.numpy as jnp
from jax.experimental import pallas as pl
from jax.experimental.pallas import tpu as pltpu
from jax.experimental.pallas import tpu_sc as plsc

_MAX_MEL = 2048
_GATHER_W = 128


def _index_kernel(dur_ref, idx_ref, len_ref):
    b = pl.program_id(0)
    d = dur_ref[0].astype(jnp.bfloat16)
    s = d.shape[0]
    row = jax.lax.broadcasted_iota(jnp.int32, (s, s), 0)
    col = jax.lax.broadcasted_iota(jnp.int32, (s, s), 1)
    lower = (col <= row).astype(jnp.bfloat16)
    csum = jax.lax.dot_general(
        lower, d, (((1,), (0,)), ((), ())),
        preferred_element_type=jnp.float32)
    total = jnp.max(csum)
    mgrid = jax.lax.broadcasted_iota(jnp.float32, (s, _MAX_MEL), 1)
    cmp = (csum <= mgrid).astype(jnp.bfloat16)
    ones = jnp.ones((1, s), jnp.bfloat16)
    cnt = jax.lax.dot_general(
        ones, cmp, (((1,), (0,)), ((), ())),
        preferred_element_type=jnp.float32)
    mrow = jax.lax.broadcasted_iota(jnp.float32, (1, _MAX_MEL), 1)
    idx = jnp.where(mrow < total, cnt, 0.0).astype(jnp.int32) + b * s
    idx_ref[0] = idx
    mel = jnp.minimum(total, float(_MAX_MEL)).astype(jnp.int32)
    len_ref[0] = jnp.broadcast_to(mel, (1, 128))


def _expand_indices(duration):
    b, s = duration.shape
    dur3 = duration.reshape(b, s, 1)
    idx, lens = pl.pallas_call(
        _index_kernel,
        grid=(b,),
        in_specs=[pl.BlockSpec((1, s, 1), lambda i: (i, 0, 0))],
        out_specs=[
            pl.BlockSpec((1, 1, _MAX_MEL), lambda i: (i, 0, 0)),
            pl.BlockSpec((1, 1, 128), lambda i: (i, 0, 0)),
        ],
        out_shape=[
            jax.ShapeDtypeStruct((b, 1, _MAX_MEL), jnp.int32),
            jax.ShapeDtypeStruct((b, 1, 128), jnp.int32),
        ],
    )(dur3)
    return idx.reshape(1, b * _MAX_MEL), lens[:, 0, 0]


def _sc_gather(x_flat, flat_idx):
    n_idx = flat_idx.shape[1]
    h = x_flat.shape[1]
    mesh = plsc.VectorSubcoreMesh(
        core_axis_name="core", subcore_axis_name="subcore")

    @functools.partial(
        pl.kernel,
        out_type=jax.ShapeDtypeStruct((n_idx, h), x_flat.dtype),
        mesh=mesh)
    def gather_kernel(x_hbm, i_hbm, o_hbm):
        def body(i_vmem, o_vmem):
            pltpu.sync_copy(x_hbm.at[i_vmem.at[0]], o_vmem)

        pltpu.emit_pipeline(
            body,
            grid=(n_idx // _GATHER_W,),
            in_specs=[pl.BlockSpec((1, _GATHER_W), lambda i: (0, i))],
            out_specs=[pl.BlockSpec((_GATHER_W, h), lambda i: (i, 0))],
            core_axis_name=("core", "subcore"),
            dimension_semantics=(pltpu.PARALLEL,),
        )(i_hbm, o_hbm)

    return gather_kernel(x_flat, flat_idx)


def kernel(x, duration, max_len):
    b, s, h = x.shape
    flat_idx, mel_len = _expand_indices(duration)
    out = _sc_gather(x.reshape(b * s, h), flat_idx)
    return out.reshape(b, _MAX_MEL, h), mel_len


# speedup vs baseline: 2661.6940x; 2661.6940x over previous
import functools

import jax
import jax.numpy as jnp
from jax.experimental import pallas as pl
from jax.experimental.pallas import tpu as pltpu
from jax.experimental.pallas import tpu_sc as plsc

_MAX_MEL = 2048
_GATHER_W = 128


def _index_kernel(dur_ref, idx_ref, len_ref):
    b = pl.program_id(0)
    d = dur_ref[0].astype(jnp.bfloat16)
    s = d.shape[0]
    row = jax.lax.broadcasted_iota(jnp.int32, (s, s), 0)
    col = jax.lax.broadcasted_iota(jnp.int32, (s, s), 1)
    lower = (col <= row).astype(jnp.bfloat16)
    csum = jax.lax.dot_general(
        lower, d, (((1,), (0,)), ((), ())),
        preferred_element_type=jnp.float32).astype(jnp.int32)
    total = jnp.max(csum)
    mgrid = jax.lax.broadcasted_iota(jnp.int32, (s, _MAX_MEL), 1)
    cmp = (csum <= mgrid).astype(jnp.bfloat16)
    ones = jnp.ones((1, s), jnp.bfloat16)
    cnt = jax.lax.dot_general(
        ones, cmp, (((1,), (0,)), ((), ())),
        preferred_element_type=jnp.float32).astype(jnp.int32)
    mrow = jax.lax.broadcasted_iota(jnp.int32, (1, _MAX_MEL), 1)
    idx = jnp.where(mrow < total, cnt, 0) + b * s
    idx_ref[0] = idx
    mel = jnp.minimum(total, _MAX_MEL)
    len_ref[0] = jnp.broadcast_to(mel, (1, 128))


def _expand_indices(duration):
    b, s = duration.shape
    dur3 = duration.reshape(b, s, 1)
    idx, lens = pl.pallas_call(
        _index_kernel,
        grid=(b,),
        in_specs=[pl.BlockSpec((1, s, 1), lambda i: (i, 0, 0))],
        out_specs=[
            pl.BlockSpec((1, 1, _MAX_MEL), lambda i: (i, 0, 0)),
            pl.BlockSpec((1, 1, 128), lambda i: (i, 0, 0)),
        ],
        out_shape=[
            jax.ShapeDtypeStruct((b, 1, _MAX_MEL), jnp.int32),
            jax.ShapeDtypeStruct((b, 1, 128), jnp.int32),
        ],
    )(dur3)
    return idx.reshape(1, b * _MAX_MEL), lens[:, 0, 0]


def _sc_gather(x_flat, flat_idx):
    n_idx = flat_idx.shape[1]
    h = x_flat.shape[1]
    mesh = plsc.VectorSubcoreMesh(
        core_axis_name="core", subcore_axis_name="subcore")

    @functools.partial(
        pl.kernel,
        out_type=jax.ShapeDtypeStruct((n_idx, h), x_flat.dtype),
        mesh=mesh)
    def gather_kernel(x_hbm, i_hbm, o_hbm):
        def body(i_vmem, o_vmem):
            pltpu.sync_copy(x_hbm.at[i_vmem.at[0]], o_vmem)

        pltpu.emit_pipeline(
            body,
            grid=(n_idx // _GATHER_W,),
            in_specs=[pl.BlockSpec((1, _GATHER_W), lambda i: (0, i))],
            out_specs=[pl.BlockSpec((_GATHER_W, h), lambda i: (i, 0))],
            core_axis_name=("core", "subcore"),
            dimension_semantics=(pltpu.PARALLEL,),
        )(i_hbm, o_hbm)

    return gather_kernel(x_flat, flat_idx)


def kernel(x, duration, max_len):
    b, s, h = x.shape
    flat_idx, mel_len = _expand_indices(duration)
    out = _sc_gather(x.reshape(b * s, h), flat_idx)
    return out.reshape(b, _MAX_MEL, h), mel_len
